# baseline (speedup 1.0000x reference)
import jax
import jax.numpy as jnp
from jax.experimental import pallas as pl
from jax.experimental.pallas import tpu as pltpu

_CHUNK = 2048
_NBUF = 8
_WCOPIES = 8


def _stream_kernel(x_hbm, w_ref, b_ref, o_hbm, xbuf, obuf, in_sem, out_sem):
    n = x_hbm.shape[0]
    nch = n // _CHUNK
    w = w_ref[...].astype(jnp.bfloat16)
    bias = b_ref[...].T

    def in_copy(i, slot):
        return pltpu.make_async_copy(
            x_hbm.at[pl.ds(i * _CHUNK, _CHUNK), :], xbuf.at[slot],
            in_sem.at[slot])

    wchunk = n // _WCOPIES

    def out_copy(j):
        return pltpu.make_async_copy(
            obuf.at[:, pl.ds(j * wchunk, wchunk)],
            o_hbm.at[:, pl.ds(j * wchunk, wchunk)], out_sem.at[j])

    for s in range(min(_NBUF, nch)):
        in_copy(s, s).start()
    for i in range(nch):
        slot = i % _NBUF
        in_copy(i, slot).wait()
        x = xbuf[slot].astype(jnp.bfloat16)
        yt = jax.lax.dot_general(w, x, (((1,), (1,)), ((), ())),
                                 preferred_element_type=jnp.float32)
        obuf[:, i * _CHUNK:(i + 1) * _CHUNK] = yt + bias
        if i + _NBUF < nch:
            in_copy(i + _NBUF, slot).start()
    for j in range(_WCOPIES):
        out_copy(j).start()
    for j in range(_WCOPIES):
        out_copy(j).wait()


def kernel(input, W, b):
    n, in_f = input.shape
    out_f = W.shape[1]
    out_t = pl.pallas_call(
        _stream_kernel,
        in_specs=[
            pl.BlockSpec(memory_space=pl.ANY),
            pl.BlockSpec(memory_space=pltpu.VMEM),
            pl.BlockSpec(memory_space=pltpu.VMEM),
        ],
        out_specs=pl.BlockSpec(memory_space=pl.ANY),
        out_shape=jax.ShapeDtypeStruct((out_f, n), jnp.float32),
        scratch_shapes=[
            pltpu.VMEM((_NBUF, _CHUNK, in_f), jnp.float32),
            pltpu.VMEM((out_f, n), jnp.float32),
            pltpu.SemaphoreType.DMA((_NBUF,)),
            pltpu.SemaphoreType.DMA((_WCOPIES,)),
        ],
    )(input, W.T, b.reshape(1, out_f))
    return out_t.T


# speedup vs baseline: 1.0564x; 1.0564x over previous
import jax
import jax.numpy as jnp
from jax.experimental import pallas as pl
from jax.experimental.pallas import tpu as pltpu

_CHUNK = 2048
_NBUF = 8


def _stream_kernel(x_hbm, w_ref, b_ref, o_hbm, xbuf, obuf, in_sem, out_sem):
    n = x_hbm.shape[0]
    nch = n // _CHUNK
    w = w_ref[...].astype(jnp.bfloat16)
    bias = b_ref[...].T

    def in_copy(i, slot):
        return pltpu.make_async_copy(
            x_hbm.at[pl.ds(i * _CHUNK, _CHUNK), :], xbuf.at[slot],
            in_sem.at[slot])

    def out_copy(i, slot):
        return pltpu.make_async_copy(
            obuf.at[slot], o_hbm.at[:, pl.ds(i * _CHUNK, _CHUNK)],
            out_sem.at[slot])

    for s in range(min(_NBUF, nch)):
        in_copy(s, s).start()
    for i in range(nch):
        slot = i % _NBUF
        in_copy(i, slot).wait()
        if i >= _NBUF:
            out_copy(i - _NBUF, slot).wait()
        x = xbuf[slot].astype(jnp.bfloat16)
        yt = jax.lax.dot_general(w, x, (((1,), (1,)), ((), ())),
                                 preferred_element_type=jnp.float32)
        obuf[slot] = yt + bias
        out_copy(i, slot).start()
        if i + _NBUF < nch:
            in_copy(i + _NBUF, slot).start()
    for i in range(max(0, nch - _NBUF), nch):
        out_copy(i, i % _NBUF).wait()


def kernel(input, W, b):
    n, in_f = input.shape
    out_f = W.shape[1]
    out_t = pl.pallas_call(
        _stream_kernel,
        in_specs=[
            pl.BlockSpec(memory_space=pl.ANY),
            pl.BlockSpec(memory_space=pltpu.VMEM),
            pl.BlockSpec(memory_space=pltpu.VMEM),
        ],
        out_specs=pl.BlockSpec(memory_space=pl.ANY),
        out_shape=jax.ShapeDtypeStruct((out_f, n), jnp.float32),
        scratch_shapes=[
            pltpu.VMEM((_NBUF, _CHUNK, in_f), jnp.float32),
            pltpu.VMEM((_NBUF, out_f, _CHUNK), jnp.float32),
            pltpu.SemaphoreType.DMA((_NBUF,)),
            pltpu.SemaphoreType.DMA((_NBUF,)),
        ],
    )(input, W.T, b.reshape(1, out_f))
    return out_t.T
